# baseline (speedup 1.0000x reference)
import jax
import jax.numpy as jnp
from jax.experimental import pallas as pl
from jax.experimental.pallas import tpu as pltpu

_A = 128
_T = 64
_R = 4096
_CHUNK = 8
_NCHUNK = _R // _CHUNK
_RG = 2
_CPI = 16
_NUM_STEP = 3
_ALPHA = 0.5
_BETA = 0.1
_REP_RADIUS = 6.0
_OFF_RADIUS = 1.0
_EPS = 1e-6


def _offroad_grad(xx, xy, nx_ref, ny_ref, pn_ref, dx_ref, dy_ref,
                  t0_ref, t1_ref):
  big = jnp.float32(3.0e38)
  act_acc = jnp.zeros((1, _A), jnp.float32)

  for rg in range(_T // _RG):
    rows = [rg * _RG + r for r in range(_RG)]
    qxb = [jnp.broadcast_to(xx[r:r + 1, :], (_CHUNK, _A)) for r in rows]
    qyb = [jnp.broadcast_to(xy[r:r + 1, :], (_CHUNK, _A)) for r in rows]

    def body(i, carry, qxb=qxb, qyb=qyb):
      out = carry
      for j in range(_CPI):
        c = i * _CPI + j
        nxc = nx_ref[c]
        nyc = ny_ref[c]
        pnc = pn_ref[c]
        dxc = dx_ref[c]
        dyc = dy_ref[c]
        new = []
        for r in range(_RG):
          sc, bnx, bny, bdx, bdy = out[r]
          s = qxb[r] * nxc + (qyb[r] * nyc + pnc)
          m = s < sc
          new.append((
              jnp.where(m, s, sc),
              jnp.where(m, nxc, bnx),
              jnp.where(m, nyc, bny),
              jnp.where(m, dxc, bdx),
              jnp.where(m, dyc, bdy),
          ))
        out = tuple(new)
      return out

    zero = jnp.zeros((_CHUNK, _A), jnp.float32)
    init = tuple((jnp.full((_CHUNK, _A), big, jnp.float32),
                  zero, zero, zero, zero) for _ in range(_RG))
    res = jax.lax.fori_loop(0, _NCHUNK // _CPI, body, init)

    for r in range(_RG):
      cur = res[r]
      for k in (4, 2, 1):
        rolled = tuple(jnp.roll(v, -k, axis=0) for v in cur)
        m = rolled[0] < cur[0]
        cur = tuple(jnp.where(m, rv, cv) for rv, cv in zip(rolled, cur))
      row = rows[r]
      bpx = -0.5 * cur[1][0:1, :]
      bpy = -0.5 * cur[2][0:1, :]
      bdxr = cur[3][0:1, :]
      bdyr = cur[4][0:1, :]
      relx = xx[row:row + 1, :] - bpx
      rely = xy[row:row + 1, :] - bpy
      dist = jnp.sqrt(relx * relx + rely * rely + 1e-12)
      cross = bdxr * rely - bdyr * relx
      sgn = jnp.sign(cross)
      active = (_OFF_RADIUS + sgn * dist) > 0.0
      coef = jnp.where(active, sgn, 0.0)
      t0_ref[row:row + 1, :] = coef * (relx / dist)
      t1_ref[row:row + 1, :] = coef * (rely / dist)
      act_acc = act_acc + jnp.where(active, 1.0, 0.0)

  return t0_ref[...], t1_ref[...], jnp.sum(act_acc)


def _repeller_grad(xx, xy):
  sx = jnp.zeros(xx.shape, jnp.float32)
  sy = jnp.zeros(xx.shape, jnp.float32)
  cnt = jnp.zeros(xx.shape, jnp.float32)
  for s in range(1, _A):
    dxv = xx - jnp.roll(xx, -s, axis=1)
    dyv = xy - jnp.roll(xy, -s, axis=1)
    d2 = dxv * dxv + (dyv * dyv + 1e-12)
    active = d2 < _REP_RADIUS * _REP_RADIUS
    inv = jnp.where(active, jax.lax.rsqrt(d2), 0.0)
    sx = sx + dxv * inv
    sy = sy + dyv * inv
    cnt = cnt + jnp.where(active, 1.0, 0.0)
  return sx, sy, jnp.sum(cnt)


def _guidance_body(xx_ref, xy_ref, px_ref, py_ref, dxb_ref, dyb_ref,
                   oxx_ref, oxy_ref, nx_ref, ny_ref, pn_ref,
                   t0_ref, t1_ref):
  pxb = px_ref[...]
  pyb = py_ref[...]
  nx_ref[...] = -2.0 * pxb
  ny_ref[...] = -2.0 * pyb
  pn_ref[...] = pxb * pxb + pyb * pyb

  xx0 = xx_ref[...]
  xy0 = xy_ref[...]
  xx, xy = xx0, xy0
  mx = jnp.zeros(xx.shape, jnp.float32)
  my = jnp.zeros(xx.shape, jnp.float32)
  vx = jnp.zeros(xx.shape, jnp.float32)
  vy = jnp.zeros(xx.shape, jnp.float32)
  b1, b2, eps_adam = 0.9, 0.999, 1e-8

  for t in range(1, _NUM_STEP + 1):
    gox, goy, cnt_off = _offroad_grad(xx, xy, nx_ref, ny_ref, pn_ref,
                                      dxb_ref, dyb_ref, t0_ref, t1_ref)
    sx, sy, cnt_rep = _repeller_grad(xx, xy)
    crep = 10.0 * (-2.0 / _REP_RADIUS) / (cnt_rep + _EPS)
    coff = 5.0 / (cnt_off + _EPS)
    gx = crep * sx + coff * gox
    gy = crep * sy + coff * goy

    mx = b1 * mx + (1.0 - b1) * gx
    my = b1 * my + (1.0 - b1) * gy
    vx = b2 * vx + (1.0 - b2) * gx * gx
    vy = b2 * vy + (1.0 - b2) * gy * gy
    c1 = 1.0 - b1 ** t
    c2 = 1.0 - b2 ** t
    xx = xx - _ALPHA * (mx / c1) / (jnp.sqrt(vx / c2) + eps_adam)
    xy = xy - _ALPHA * (my / c1) / (jnp.sqrt(vy / c2) + eps_adam)
    xx = xx0 + jnp.clip(xx - xx0, -_BETA, _BETA)
    xy = xy0 + jnp.clip(xy - xy0, -_BETA, _BETA)

  oxx_ref[...] = xx
  oxy_ref[...] = xy


@jax.jit
def kernel(x, road_xyz, road_dir):
  xx = x[:, :, 0].T
  xy = x[:, :, 1].T
  shape3 = (_NCHUNK, _CHUNK, 1)
  full3 = (_NCHUNK, _CHUNK, _A)
  pxb = jnp.broadcast_to(road_xyz[:, 0].reshape(shape3), full3)
  pyb = jnp.broadcast_to(road_xyz[:, 1].reshape(shape3), full3)
  dxb = jnp.broadcast_to(road_dir[:, 0].reshape(shape3), full3)
  dyb = jnp.broadcast_to(road_dir[:, 1].reshape(shape3), full3)
  oxx, oxy = pl.pallas_call(
      _guidance_body,
      out_shape=(
          jax.ShapeDtypeStruct((_T, _A), jnp.float32),
          jax.ShapeDtypeStruct((_T, _A), jnp.float32),
      ),
      scratch_shapes=[
          pltpu.VMEM(full3, jnp.float32),
          pltpu.VMEM(full3, jnp.float32),
          pltpu.VMEM(full3, jnp.float32),
          pltpu.VMEM((_T, _A), jnp.float32),
          pltpu.VMEM((_T, _A), jnp.float32),
      ],
  )(xx, xy, pxb, pyb, dxb, dyb)
  return jnp.stack([oxx.T, oxy.T], axis=-1)


# speedup vs baseline: 1.0745x; 1.0745x over previous
import jax
import jax.numpy as jnp
from jax.experimental import pallas as pl
from jax.experimental.pallas import tpu as pltpu

_A = 128
_T = 64
_R = 4096
_CHUNK = 8
_NCHUNK = _R // _CHUNK
_RG = 4
_CPI = 64
_NUM_STEP = 3
_ALPHA = 0.5
_BETA = 0.1
_REP_RADIUS = 6.0
_OFF_RADIUS = 1.0
_EPS = 1e-6


def _offroad_grad(xx, xy, nx_ref, ny_ref, pn_ref, dx_ref, dy_ref,
                  t0_ref, t1_ref):
  big = jnp.float32(3.0e38)
  act_acc = jnp.zeros((1, _A), jnp.float32)

  for rg in range(_T // _RG):
    rows = [rg * _RG + r for r in range(_RG)]
    qxb = [jnp.broadcast_to(xx[r:r + 1, :], (_CHUNK, _A)) for r in rows]
    qyb = [jnp.broadcast_to(xy[r:r + 1, :], (_CHUNK, _A)) for r in rows]

    def body(i, carry, qxb=qxb, qyb=qyb):
      out = carry
      for j in range(_CPI):
        c = i * _CPI + j
        nxc = nx_ref[c]
        nyc = ny_ref[c]
        pnc = pn_ref[c]
        dxc = dx_ref[c]
        dyc = dy_ref[c]
        new = []
        for r in range(_RG):
          sc, bnx, bny, bdx, bdy = out[r]
          s = qxb[r] * nxc + (qyb[r] * nyc + pnc)
          m = s < sc
          new.append((
              jnp.where(m, s, sc),
              jnp.where(m, nxc, bnx),
              jnp.where(m, nyc, bny),
              jnp.where(m, dxc, bdx),
              jnp.where(m, dyc, bdy),
          ))
        out = tuple(new)
      return out

    zero = jnp.zeros((_CHUNK, _A), jnp.float32)
    init = tuple((jnp.full((_CHUNK, _A), big, jnp.float32),
                  zero, zero, zero, zero) for _ in range(_RG))
    res = jax.lax.fori_loop(0, _NCHUNK // _CPI, body, init)

    for r in range(_RG):
      cur = res[r]
      for k in (4, 2, 1):
        rolled = tuple(jnp.roll(v, -k, axis=0) for v in cur)
        m = rolled[0] < cur[0]
        cur = tuple(jnp.where(m, rv, cv) for rv, cv in zip(rolled, cur))
      row = rows[r]
      bpx = -0.5 * cur[1][0:1, :]
      bpy = -0.5 * cur[2][0:1, :]
      bdxr = cur[3][0:1, :]
      bdyr = cur[4][0:1, :]
      relx = xx[row:row + 1, :] - bpx
      rely = xy[row:row + 1, :] - bpy
      dist = jnp.sqrt(relx * relx + rely * rely + 1e-12)
      cross = bdxr * rely - bdyr * relx
      sgn = jnp.sign(cross)
      active = (_OFF_RADIUS + sgn * dist) > 0.0
      coef = jnp.where(active, sgn, 0.0)
      t0_ref[row:row + 1, :] = coef * (relx / dist)
      t1_ref[row:row + 1, :] = coef * (rely / dist)
      act_acc = act_acc + jnp.where(active, 1.0, 0.0)

  return t0_ref[...], t1_ref[...], jnp.sum(act_acc)


def _repeller_grad(xx, xy):
  sx = jnp.zeros(xx.shape, jnp.float32)
  sy = jnp.zeros(xx.shape, jnp.float32)
  cnt = jnp.zeros(xx.shape, jnp.float32)
  for s in range(1, _A):
    dxv = xx - jnp.roll(xx, -s, axis=1)
    dyv = xy - jnp.roll(xy, -s, axis=1)
    d2 = dxv * dxv + (dyv * dyv + 1e-12)
    active = d2 < _REP_RADIUS * _REP_RADIUS
    inv = jnp.where(active, jax.lax.rsqrt(d2), 0.0)
    sx = sx + dxv * inv
    sy = sy + dyv * inv
    cnt = cnt + jnp.where(active, 1.0, 0.0)
  return sx, sy, jnp.sum(cnt)


def _guidance_body(xx_ref, xy_ref, px_ref, py_ref, dxb_ref, dyb_ref,
                   oxx_ref, oxy_ref, nx_ref, ny_ref, pn_ref,
                   t0_ref, t1_ref):
  pxb = px_ref[...]
  pyb = py_ref[...]
  nx_ref[...] = -2.0 * pxb
  ny_ref[...] = -2.0 * pyb
  pn_ref[...] = pxb * pxb + pyb * pyb

  xx0 = xx_ref[...]
  xy0 = xy_ref[...]
  xx, xy = xx0, xy0
  mx = jnp.zeros(xx.shape, jnp.float32)
  my = jnp.zeros(xx.shape, jnp.float32)
  vx = jnp.zeros(xx.shape, jnp.float32)
  vy = jnp.zeros(xx.shape, jnp.float32)
  b1, b2, eps_adam = 0.9, 0.999, 1e-8

  for t in range(1, _NUM_STEP + 1):
    gox, goy, cnt_off = _offroad_grad(xx, xy, nx_ref, ny_ref, pn_ref,
                                      dxb_ref, dyb_ref, t0_ref, t1_ref)
    sx, sy, cnt_rep = _repeller_grad(xx, xy)
    crep = 10.0 * (-2.0 / _REP_RADIUS) / (cnt_rep + _EPS)
    coff = 5.0 / (cnt_off + _EPS)
    gx = crep * sx + coff * gox
    gy = crep * sy + coff * goy

    mx = b1 * mx + (1.0 - b1) * gx
    my = b1 * my + (1.0 - b1) * gy
    vx = b2 * vx + (1.0 - b2) * gx * gx
    vy = b2 * vy + (1.0 - b2) * gy * gy
    c1 = 1.0 - b1 ** t
    c2 = 1.0 - b2 ** t
    xx = xx - _ALPHA * (mx / c1) / (jnp.sqrt(vx / c2) + eps_adam)
    xy = xy - _ALPHA * (my / c1) / (jnp.sqrt(vy / c2) + eps_adam)
    xx = xx0 + jnp.clip(xx - xx0, -_BETA, _BETA)
    xy = xy0 + jnp.clip(xy - xy0, -_BETA, _BETA)

  oxx_ref[...] = xx
  oxy_ref[...] = xy


@jax.jit
def kernel(x, road_xyz, road_dir):
  xx = x[:, :, 0].T
  xy = x[:, :, 1].T
  shape3 = (_NCHUNK, _CHUNK, 1)
  full3 = (_NCHUNK, _CHUNK, _A)
  pxb = jnp.broadcast_to(road_xyz[:, 0].reshape(shape3), full3)
  pyb = jnp.broadcast_to(road_xyz[:, 1].reshape(shape3), full3)
  dxb = jnp.broadcast_to(road_dir[:, 0].reshape(shape3), full3)
  dyb = jnp.broadcast_to(road_dir[:, 1].reshape(shape3), full3)
  oxx, oxy = pl.pallas_call(
      _guidance_body,
      out_shape=(
          jax.ShapeDtypeStruct((_T, _A), jnp.float32),
          jax.ShapeDtypeStruct((_T, _A), jnp.float32),
      ),
      scratch_shapes=[
          pltpu.VMEM(full3, jnp.float32),
          pltpu.VMEM(full3, jnp.float32),
          pltpu.VMEM(full3, jnp.float32),
          pltpu.VMEM((_T, _A), jnp.float32),
          pltpu.VMEM((_T, _A), jnp.float32),
      ],
  )(xx, xy, pxb, pyb, dxb, dyb)
  return jnp.stack([oxx.T, oxy.T], axis=-1)


# speedup vs baseline: 1.0777x; 1.0030x over previous
import jax
import jax.numpy as jnp
from jax.experimental import pallas as pl
from jax.experimental.pallas import tpu as pltpu

_A = 128
_T = 64
_R = 4096
_CHUNK = 8
_NCHUNK = _R // _CHUNK
_RG = 4
_CPI = 128
_NUM_STEP = 3
_ALPHA = 0.5
_BETA = 0.1
_REP_RADIUS = 6.0
_OFF_RADIUS = 1.0
_EPS = 1e-6


def _offroad_grad(xx, xy, nx_ref, ny_ref, pn_ref, dx_ref, dy_ref,
                  t0_ref, t1_ref):
  big = jnp.float32(3.0e38)
  act_acc = jnp.zeros((1, _A), jnp.float32)

  for rg in range(_T // _RG):
    rows = [rg * _RG + r for r in range(_RG)]
    qxb = [jnp.broadcast_to(xx[r:r + 1, :], (_CHUNK, _A)) for r in rows]
    qyb = [jnp.broadcast_to(xy[r:r + 1, :], (_CHUNK, _A)) for r in rows]

    def body(i, carry, qxb=qxb, qyb=qyb):
      out = carry
      for j in range(_CPI):
        c = i * _CPI + j
        nxc = nx_ref[c]
        nyc = ny_ref[c]
        pnc = pn_ref[c]
        dxc = dx_ref[c]
        dyc = dy_ref[c]
        new = []
        for r in range(_RG):
          sc, bnx, bny, bdx, bdy = out[r]
          s = qxb[r] * nxc + (qyb[r] * nyc + pnc)
          m = s < sc
          new.append((
              jnp.where(m, s, sc),
              jnp.where(m, nxc, bnx),
              jnp.where(m, nyc, bny),
              jnp.where(m, dxc, bdx),
              jnp.where(m, dyc, bdy),
          ))
        out = tuple(new)
      return out

    zero = jnp.zeros((_CHUNK, _A), jnp.float32)
    init = tuple((jnp.full((_CHUNK, _A), big, jnp.float32),
                  zero, zero, zero, zero) for _ in range(_RG))
    res = jax.lax.fori_loop(0, _NCHUNK // _CPI, body, init)

    for r in range(_RG):
      cur = res[r]
      for k in (4, 2, 1):
        rolled = tuple(jnp.roll(v, -k, axis=0) for v in cur)
        m = rolled[0] < cur[0]
        cur = tuple(jnp.where(m, rv, cv) for rv, cv in zip(rolled, cur))
      row = rows[r]
      bpx = -0.5 * cur[1][0:1, :]
      bpy = -0.5 * cur[2][0:1, :]
      bdxr = cur[3][0:1, :]
      bdyr = cur[4][0:1, :]
      relx = xx[row:row + 1, :] - bpx
      rely = xy[row:row + 1, :] - bpy
      dist = jnp.sqrt(relx * relx + rely * rely + 1e-12)
      cross = bdxr * rely - bdyr * relx
      sgn = jnp.sign(cross)
      active = (_OFF_RADIUS + sgn * dist) > 0.0
      coef = jnp.where(active, sgn, 0.0)
      t0_ref[row:row + 1, :] = coef * (relx / dist)
      t1_ref[row:row + 1, :] = coef * (rely / dist)
      act_acc = act_acc + jnp.where(active, 1.0, 0.0)

  return t0_ref[...], t1_ref[...], jnp.sum(act_acc)


def _repeller_grad(xx, xy):
  sx = jnp.zeros(xx.shape, jnp.float32)
  sy = jnp.zeros(xx.shape, jnp.float32)
  cnt = jnp.zeros(xx.shape, jnp.float32)
  for s in range(1, _A):
    dxv = xx - jnp.roll(xx, -s, axis=1)
    dyv = xy - jnp.roll(xy, -s, axis=1)
    d2 = dxv * dxv + (dyv * dyv + 1e-12)
    active = d2 < _REP_RADIUS * _REP_RADIUS
    inv = jnp.where(active, jax.lax.rsqrt(d2), 0.0)
    sx = sx + dxv * inv
    sy = sy + dyv * inv
    cnt = cnt + jnp.where(active, 1.0, 0.0)
  return sx, sy, jnp.sum(cnt)


def _guidance_body(xx_ref, xy_ref, px_ref, py_ref, dxb_ref, dyb_ref,
                   oxx_ref, oxy_ref, nx_ref, ny_ref, pn_ref,
                   t0_ref, t1_ref):
  pxb = px_ref[...]
  pyb = py_ref[...]
  nx_ref[...] = -2.0 * pxb
  ny_ref[...] = -2.0 * pyb
  pn_ref[...] = pxb * pxb + pyb * pyb

  xx0 = xx_ref[...]
  xy0 = xy_ref[...]
  xx, xy = xx0, xy0
  mx = jnp.zeros(xx.shape, jnp.float32)
  my = jnp.zeros(xx.shape, jnp.float32)
  vx = jnp.zeros(xx.shape, jnp.float32)
  vy = jnp.zeros(xx.shape, jnp.float32)
  b1, b2, eps_adam = 0.9, 0.999, 1e-8

  for t in range(1, _NUM_STEP + 1):
    gox, goy, cnt_off = _offroad_grad(xx, xy, nx_ref, ny_ref, pn_ref,
                                      dxb_ref, dyb_ref, t0_ref, t1_ref)
    sx, sy, cnt_rep = _repeller_grad(xx, xy)
    crep = 10.0 * (-2.0 / _REP_RADIUS) / (cnt_rep + _EPS)
    coff = 5.0 / (cnt_off + _EPS)
    gx = crep * sx + coff * gox
    gy = crep * sy + coff * goy

    mx = b1 * mx + (1.0 - b1) * gx
    my = b1 * my + (1.0 - b1) * gy
    vx = b2 * vx + (1.0 - b2) * gx * gx
    vy = b2 * vy + (1.0 - b2) * gy * gy
    c1 = 1.0 - b1 ** t
    c2 = 1.0 - b2 ** t
    xx = xx - _ALPHA * (mx / c1) / (jnp.sqrt(vx / c2) + eps_adam)
    xy = xy - _ALPHA * (my / c1) / (jnp.sqrt(vy / c2) + eps_adam)
    xx = xx0 + jnp.clip(xx - xx0, -_BETA, _BETA)
    xy = xy0 + jnp.clip(xy - xy0, -_BETA, _BETA)

  oxx_ref[...] = xx
  oxy_ref[...] = xy


@jax.jit
def kernel(x, road_xyz, road_dir):
  xx = x[:, :, 0].T
  xy = x[:, :, 1].T
  shape3 = (_NCHUNK, _CHUNK, 1)
  full3 = (_NCHUNK, _CHUNK, _A)
  pxb = jnp.broadcast_to(road_xyz[:, 0].reshape(shape3), full3)
  pyb = jnp.broadcast_to(road_xyz[:, 1].reshape(shape3), full3)
  dxb = jnp.broadcast_to(road_dir[:, 0].reshape(shape3), full3)
  dyb = jnp.broadcast_to(road_dir[:, 1].reshape(shape3), full3)
  oxx, oxy = pl.pallas_call(
      _guidance_body,
      out_shape=(
          jax.ShapeDtypeStruct((_T, _A), jnp.float32),
          jax.ShapeDtypeStruct((_T, _A), jnp.float32),
      ),
      scratch_shapes=[
          pltpu.VMEM(full3, jnp.float32),
          pltpu.VMEM(full3, jnp.float32),
          pltpu.VMEM(full3, jnp.float32),
          pltpu.VMEM((_T, _A), jnp.float32),
          pltpu.VMEM((_T, _A), jnp.float32),
      ],
  )(xx, xy, pxb, pyb, dxb, dyb)
  return jnp.stack([oxx.T, oxy.T], axis=-1)


# speedup vs baseline: 1.0792x; 1.0014x over previous
import jax
import jax.numpy as jnp
from jax.experimental import pallas as pl
from jax.experimental.pallas import tpu as pltpu

_A = 128
_T = 64
_R = 4096
_CHUNK = 8
_NCHUNK = _R // _CHUNK
_RG = 4
_CPI = 128
_NUM_STEP = 3
_ALPHA = 0.5
_BETA = 0.1
_REP_RADIUS = 6.0
_OFF_RADIUS = 1.0
_EPS = 1e-6


def _offroad_grad(xx, xy, nx_ref, ny_ref, pn_ref, dx_ref, dy_ref,
                  t0_ref, t1_ref):
  big = jnp.float32(3.0e38)
  act_acc = jnp.zeros((1, _A), jnp.float32)

  for rg in range(_T // _RG):
    rows = [rg * _RG + r for r in range(_RG)]
    qxb = [jnp.broadcast_to(xx[r:r + 1, :], (_CHUNK, _A)) for r in rows]
    qyb = [jnp.broadcast_to(xy[r:r + 1, :], (_CHUNK, _A)) for r in rows]

    def body(i, carry, qxb=qxb, qyb=qyb):
      out = carry
      for j in range(_CPI):
        c = i * _CPI + j
        nxc = nx_ref[c]
        nyc = ny_ref[c]
        pnc = pn_ref[c]
        dxc = dx_ref[c]
        dyc = dy_ref[c]
        new = []
        for r in range(_RG):
          sc, bnx, bny, bdx, bdy = out[r]
          s = qxb[r] * nxc + (qyb[r] * nyc + pnc)
          m = s < sc
          new.append((
              jnp.minimum(s, sc),
              jnp.where(m, nxc, bnx),
              jnp.where(m, nyc, bny),
              jnp.where(m, dxc, bdx),
              jnp.where(m, dyc, bdy),
          ))
        out = tuple(new)
      return out

    zero = jnp.zeros((_CHUNK, _A), jnp.float32)
    init = tuple((jnp.full((_CHUNK, _A), big, jnp.float32),
                  zero, zero, zero, zero) for _ in range(_RG))
    res = jax.lax.fori_loop(0, _NCHUNK // _CPI, body, init)

    for r in range(_RG):
      cur = res[r]
      for k in (4, 2, 1):
        rolled = tuple(jnp.roll(v, -k, axis=0) for v in cur)
        m = rolled[0] < cur[0]
        cur = (jnp.minimum(rolled[0], cur[0]),) + tuple(
            jnp.where(m, rv, cv) for rv, cv in zip(rolled[1:], cur[1:]))
      row = rows[r]
      bpx = -0.5 * cur[1][0:1, :]
      bpy = -0.5 * cur[2][0:1, :]
      bdxr = cur[3][0:1, :]
      bdyr = cur[4][0:1, :]
      relx = xx[row:row + 1, :] - bpx
      rely = xy[row:row + 1, :] - bpy
      dist = jnp.sqrt(relx * relx + rely * rely + 1e-12)
      cross = bdxr * rely - bdyr * relx
      sgn = jnp.sign(cross)
      active = (_OFF_RADIUS + sgn * dist) > 0.0
      coef = jnp.where(active, sgn, 0.0)
      t0_ref[row:row + 1, :] = coef * (relx / dist)
      t1_ref[row:row + 1, :] = coef * (rely / dist)
      act_acc = act_acc + jnp.where(active, 1.0, 0.0)

  return t0_ref[...], t1_ref[...], jnp.sum(act_acc)


def _repeller_grad(xx, xy):
  sx = jnp.zeros(xx.shape, jnp.float32)
  sy = jnp.zeros(xx.shape, jnp.float32)
  cnt = jnp.zeros(xx.shape, jnp.float32)
  for s in range(1, _A):
    dxv = xx - jnp.roll(xx, -s, axis=1)
    dyv = xy - jnp.roll(xy, -s, axis=1)
    d2 = dxv * dxv + (dyv * dyv + 1e-12)
    active = d2 < _REP_RADIUS * _REP_RADIUS
    inv = jnp.where(active, jax.lax.rsqrt(d2), 0.0)
    sx = sx + dxv * inv
    sy = sy + dyv * inv
    cnt = cnt + jnp.where(active, 1.0, 0.0)
  return sx, sy, jnp.sum(cnt)


def _guidance_body(xx_ref, xy_ref, px_ref, py_ref, dxb_ref, dyb_ref,
                   oxx_ref, oxy_ref, nx_ref, ny_ref, pn_ref,
                   t0_ref, t1_ref):
  pxb = px_ref[...]
  pyb = py_ref[...]
  nx_ref[...] = -2.0 * pxb
  ny_ref[...] = -2.0 * pyb
  pn_ref[...] = pxb * pxb + pyb * pyb

  xx0 = xx_ref[...]
  xy0 = xy_ref[...]
  xx, xy = xx0, xy0
  mx = jnp.zeros(xx.shape, jnp.float32)
  my = jnp.zeros(xx.shape, jnp.float32)
  vx = jnp.zeros(xx.shape, jnp.float32)
  vy = jnp.zeros(xx.shape, jnp.float32)
  b1, b2, eps_adam = 0.9, 0.999, 1e-8

  for t in range(1, _NUM_STEP + 1):
    gox, goy, cnt_off = _offroad_grad(xx, xy, nx_ref, ny_ref, pn_ref,
                                      dxb_ref, dyb_ref, t0_ref, t1_ref)
    sx, sy, cnt_rep = _repeller_grad(xx, xy)
    crep = 10.0 * (-2.0 / _REP_RADIUS) / (cnt_rep + _EPS)
    coff = 5.0 / (cnt_off + _EPS)
    gx = crep * sx + coff * gox
    gy = crep * sy + coff * goy

    mx = b1 * mx + (1.0 - b1) * gx
    my = b1 * my + (1.0 - b1) * gy
    vx = b2 * vx + (1.0 - b2) * gx * gx
    vy = b2 * vy + (1.0 - b2) * gy * gy
    c1 = 1.0 - b1 ** t
    c2 = 1.0 - b2 ** t
    xx = xx - _ALPHA * (mx / c1) / (jnp.sqrt(vx / c2) + eps_adam)
    xy = xy - _ALPHA * (my / c1) / (jnp.sqrt(vy / c2) + eps_adam)
    xx = xx0 + jnp.clip(xx - xx0, -_BETA, _BETA)
    xy = xy0 + jnp.clip(xy - xy0, -_BETA, _BETA)

  oxx_ref[...] = xx
  oxy_ref[...] = xy


@jax.jit
def kernel(x, road_xyz, road_dir):
  xx = x[:, :, 0].T
  xy = x[:, :, 1].T
  shape3 = (_NCHUNK, _CHUNK, 1)
  full3 = (_NCHUNK, _CHUNK, _A)
  pxb = jnp.broadcast_to(road_xyz[:, 0].reshape(shape3), full3)
  pyb = jnp.broadcast_to(road_xyz[:, 1].reshape(shape3), full3)
  dxb = jnp.broadcast_to(road_dir[:, 0].reshape(shape3), full3)
  dyb = jnp.broadcast_to(road_dir[:, 1].reshape(shape3), full3)
  oxx, oxy = pl.pallas_call(
      _guidance_body,
      out_shape=(
          jax.ShapeDtypeStruct((_T, _A), jnp.float32),
          jax.ShapeDtypeStruct((_T, _A), jnp.float32),
      ),
      scratch_shapes=[
          pltpu.VMEM(full3, jnp.float32),
          pltpu.VMEM(full3, jnp.float32),
          pltpu.VMEM(full3, jnp.float32),
          pltpu.VMEM((_T, _A), jnp.float32),
          pltpu.VMEM((_T, _A), jnp.float32),
      ],
  )(xx, xy, pxb, pyb, dxb, dyb)
  return jnp.stack([oxx.T, oxy.T], axis=-1)
